# baseline (speedup 1.0000x reference)
import functools

import jax
import jax.numpy as jnp
from jax.experimental import pallas as pl
from jax.experimental.pallas import tpu as pltpu

_D_MODEL = 2048
_NUM_SLOTS = 1024
_MAX_K = 8
_NEG = -1e30


def _dpsn_block(x_ref, rw_ref, pool_ref, out_ref, aux_ref, s_buf,
                acc_probs, acc_mask, *, num_blocks, total_tokens):
    i = pl.program_id(0)
    pool = pool_ref[...]

    out_ref[...] = jax.lax.dot_general(
        s_buf[...], pool, (((1,), (0,)), ((), ())),
        preferred_element_type=jnp.float32)

    x = x_ref[...]
    rw = rw_ref[...]
    logits = jax.lax.dot_general(x, rw, (((1,), (0,)), ((), ())),
                                 preferred_element_type=jnp.float32)
    p_all = jax.lax.dot_general(x, pool, (((1,), (1,)), ((), ())),
                                preferred_element_type=jnp.float32)

    m1 = jnp.max(logits, axis=-1, keepdims=True)
    e = jnp.exp(logits - m1)
    denom_full = jnp.sum(e, axis=-1, keepdims=True)

    avail = logits
    t = m1
    for _ in range(_MAX_K - 1):
        avail = jnp.where(avail >= t, _NEG, avail)
        t = jnp.max(avail, axis=-1, keepdims=True)
    maskf = (logits >= t).astype(jnp.float32)

    e_top = e * maskf
    denom_top = jnp.sum(e_top, axis=-1, keepdims=True)
    w_dense = e_top * (1.0 / denom_top)

    s_buf[...] = w_dense * p_all

    probs_sum = jnp.sum(e * (1.0 / denom_full), axis=0, keepdims=True)
    mask_sum = jnp.sum(maskf, axis=0, keepdims=True)

    @pl.when(i == 0)
    def _init():
        acc_probs[...] = jnp.zeros_like(acc_probs)
        acc_mask[...] = jnp.zeros_like(acc_mask)

    @pl.when(i < num_blocks)
    def _acc():
        acc_probs[...] += probs_sum
        acc_mask[...] += mask_sum

    @pl.when(i == num_blocks)
    def _finish():
        inv_t = jnp.float32(1.0 / total_tokens)
        aux = _NUM_SLOTS * jnp.sum((acc_mask[...] * inv_t) *
                                   (acc_probs[...] * inv_t))
        aux_ref[...] = jnp.full((8, 128), aux, jnp.float32)


@functools.partial(jax.jit, static_argnames=("interpret",))
def _dpsn(x, param_pool, router_w, interpret=False):
    b, s, d = x.shape
    tokens = b * s
    xf = x.reshape(tokens, d)
    tb = 512
    num_blocks = tokens // tb

    out, aux = pl.pallas_call(
        functools.partial(_dpsn_block, num_blocks=num_blocks,
                          total_tokens=tokens),
        grid=(num_blocks + 1,),
        in_specs=[
            pl.BlockSpec((tb, d), lambda i: (jnp.minimum(i, num_blocks - 1), 0)),
            pl.BlockSpec((d, _NUM_SLOTS), lambda i: (0, 0)),
            pl.BlockSpec((_NUM_SLOTS, d), lambda i: (0, 0)),
        ],
        out_specs=[
            pl.BlockSpec((tb, d), lambda i: (jnp.maximum(i - 1, 0), 0)),
            pl.BlockSpec((8, 128), lambda i: (0, 0)),
        ],
        out_shape=[
            jax.ShapeDtypeStruct((tokens, d), jnp.float32),
            jax.ShapeDtypeStruct((8, 128), jnp.float32),
        ],
        scratch_shapes=[
            pltpu.VMEM((tb, _NUM_SLOTS), jnp.float32),
            pltpu.VMEM((1, _NUM_SLOTS), jnp.float32),
            pltpu.VMEM((1, _NUM_SLOTS), jnp.float32),
        ],
        interpret=interpret,
    )(xf, router_w, param_pool)
    return out.reshape(b, s, d), aux[0, 0]


def kernel(x, param_pool, router_w):
    return _dpsn(x, param_pool, router_w)


# speedup vs baseline: 1.0009x; 1.0009x over previous
import functools

import jax
import jax.numpy as jnp
from jax.experimental import pallas as pl
from jax.experimental.pallas import tpu as pltpu

_D_MODEL = 2048
_NUM_SLOTS = 1024
_MAX_K = 8
_NEG = -1e30


def _dpsn_block(x_ref, rw_ref, pool_ref, out_ref, aux_ref, s_buf,
                acc_probs, acc_mask, *, num_blocks, total_tokens):
    i = pl.program_id(0)
    pool = pool_ref[...]

    out_ref[...] = jax.lax.dot_general(
        s_buf[...], pool, (((1,), (0,)), ((), ())),
        preferred_element_type=jnp.float32)

    x = x_ref[...]
    rw = rw_ref[...]
    logits = jax.lax.dot_general(x, rw, (((1,), (0,)), ((), ())),
                                 preferred_element_type=jnp.float32)
    p_all = jax.lax.dot_general(x, pool, (((1,), (1,)), ((), ())),
                                preferred_element_type=jnp.float32)

    m1 = jnp.max(logits, axis=-1, keepdims=True)
    e = jnp.exp(logits - m1)
    denom_full = jnp.sum(e, axis=-1, keepdims=True)

    avail = logits
    t = m1
    for _ in range(_MAX_K - 1):
        avail = jnp.where(avail >= t, _NEG, avail)
        t = jnp.max(avail, axis=-1, keepdims=True)
    maskf = (logits >= t).astype(jnp.float32)

    e_top = e * maskf
    denom_top = jnp.sum(e_top, axis=-1, keepdims=True)
    w_dense = e_top * (1.0 / denom_top)

    s_buf[...] = w_dense * p_all

    probs_sum = jnp.sum(e * (1.0 / denom_full), axis=0, keepdims=True)
    mask_sum = jnp.sum(maskf, axis=0, keepdims=True)

    @pl.when(i == 0)
    def _init():
        acc_probs[...] = jnp.zeros_like(acc_probs)
        acc_mask[...] = jnp.zeros_like(acc_mask)

    @pl.when(i < num_blocks)
    def _acc():
        acc_probs[...] += probs_sum
        acc_mask[...] += mask_sum

    @pl.when(i == num_blocks)
    def _finish():
        inv_t = jnp.float32(1.0 / total_tokens)
        aux = _NUM_SLOTS * jnp.sum((acc_mask[...] * inv_t) *
                                   (acc_probs[...] * inv_t))
        aux_ref[...] = jnp.full((8, 128), aux, jnp.float32)


@functools.partial(jax.jit, static_argnames=("interpret",))
def _dpsn(x, param_pool, router_w, interpret=False):
    b, s, d = x.shape
    tokens = b * s
    xf = x.reshape(tokens, d)
    tb = 256
    num_blocks = tokens // tb

    out, aux = pl.pallas_call(
        functools.partial(_dpsn_block, num_blocks=num_blocks,
                          total_tokens=tokens),
        grid=(num_blocks + 1,),
        in_specs=[
            pl.BlockSpec((tb, d), lambda i: (jnp.minimum(i, num_blocks - 1), 0)),
            pl.BlockSpec((d, _NUM_SLOTS), lambda i: (0, 0)),
            pl.BlockSpec((_NUM_SLOTS, d), lambda i: (0, 0)),
        ],
        out_specs=[
            pl.BlockSpec((tb, d), lambda i: (jnp.maximum(i - 1, 0), 0)),
            pl.BlockSpec((8, 128), lambda i: (0, 0)),
        ],
        out_shape=[
            jax.ShapeDtypeStruct((tokens, d), jnp.float32),
            jax.ShapeDtypeStruct((8, 128), jnp.float32),
        ],
        scratch_shapes=[
            pltpu.VMEM((tb, _NUM_SLOTS), jnp.float32),
            pltpu.VMEM((1, _NUM_SLOTS), jnp.float32),
            pltpu.VMEM((1, _NUM_SLOTS), jnp.float32),
        ],
        interpret=interpret,
    )(xf, router_w, param_pool)
    return out.reshape(b, s, d), aux[0, 0]


def kernel(x, param_pool, router_w):
    return _dpsn(x, param_pool, router_w)
